# Initial kernel scaffold; baseline (speedup 1.0000x reference)
import jax
import jax.numpy as jnp
from jax.experimental import pallas as pl


def kernel(x, edge_index, W1, b1, W2, b2, Wih_f, Whh_f, bih_f, bhh_f, Wih_b, Whh_b, bih_b, bhh_b, Watt, batt, Wlin, blin, Wn1, bn1, Wn2, bn2, We1, be1, We2, be2):
    raise NotImplementedError("write your pallas kernel here")


# speedup vs baseline: 6.2220x; 6.2220x over previous
import functools

import jax
import jax.numpy as jnp
from jax import lax
from jax.experimental import pallas as pl
from jax.experimental.pallas import tpu as pltpu, tpu_sc as plsc

N = 50000
E = 800000
H = 64
HALF = 25000
RPAD = 25088
TRASH = 25000
NPAD = 2 * RPAD
TPC = 1568
EPT = E // 16
EPW = E // 32

_MESH = plsc.VectorSubcoreMesh(core_axis_name="c", subcore_axis_name="s",
                               num_cores=2, num_subcores=16)

F32 = jnp.float32


def _zero16(ref, n16):
    for j in range(n16):
        ref[pl.ds(j * 16, 16)] = jnp.zeros((16,), F32)


def _ones16(ref, n16):
    for j in range(n16):
        ref[pl.ds(j * 16, 16)] = jnp.full((16,), 1.0, F32)


@functools.partial(
    pl.kernel,
    out_type=[jax.ShapeDtypeStruct((NPAD,), F32),
              jax.ShapeDtypeStruct((NPAD,), F32)],
    mesh=_MESH,
    compiler_params=pltpu.CompilerParams(use_tc_tiling_on_sc=False),
    scratch_types=[
        pltpu.VMEM((128,), jnp.int32),
        pltpu.VMEM((128,), jnp.int32),
        pltpu.VMEM((128,), jnp.int32),
        pltpu.VMEM((80,), jnp.int32),
        pltpu.VMEM((80,), jnp.int32),
        pltpu.VMEM((80,), jnp.int32),
        pltpu.VMEM((128,), F32),
        pltpu.VMEM((80,), F32),
        pltpu.VMEM((128,), F32),
        pltpu.VMEM((80,), F32),
        pltpu.VMEM((16,), F32),
        pltpu.VMEM((TPC,), F32),
        pltpu.VMEM_SHARED((RPAD,), F32),
        pltpu.VMEM_SHARED((RPAD,), F32),
    ],
)
def _sc_deg(row_hbm, col_hbm, cnt_hbm, lc_hbm,
            rowv, colv, lrowv, rowt, colt, lrowt,
            ones128, ones80, eq128, eq80, zb, obuf, acc_c, acc_l):
    cid = lax.axis_index("c")
    sid = lax.axis_index("s")
    base_n = cid * HALF
    _ones16(ones128, 8)
    _ones16(ones80, 5)
    _zero16(zb, 1)

    def zloop(i, carry):
        off = sid * TPC + i * 16
        pltpu.sync_copy(zb, acc_c.at[pl.ds(off, 16)])
        pltpu.sync_copy(zb, acc_l.at[pl.ds(off, 16)])
        return carry

    lax.fori_loop(0, TPC // 16, zloop, 0)
    plsc.subcore_barrier()

    base_e = sid * EPT

    def idx_math(rsrc, csrc, ldst, edst, n16):
        for j in range(n16):
            rv = rsrc[pl.ds(j * 16, 16)]
            cv = csrc[pl.ds(j * 16, 16)]
            lv = rv - base_n
            oob = jnp.logical_or(lv < 0, lv >= HALF)
            ldst[pl.ds(j * 16, 16)] = jnp.where(oob, TRASH, lv)
            edst[pl.ds(j * 16, 16)] = jnp.where(rv == cv,
                                                jnp.full((16,), 1.0, F32),
                                                jnp.zeros((16,), F32))

    def chunk(ci, carry):
        off = base_e + ci * 128
        pltpu.sync_copy(row_hbm.at[pl.ds(off, 128)], rowv)
        pltpu.sync_copy(col_hbm.at[pl.ds(off, 128)], colv)
        idx_math(rowv, colv, lrowv, eq128, 8)
        pltpu.sync_copy(ones128, acc_c.at[lrowv], add=True)
        pltpu.sync_copy(eq128, acc_l.at[lrowv], add=True)
        return carry

    lax.fori_loop(0, 390, chunk, 0)

    offt = base_e + 390 * 128
    pltpu.sync_copy(row_hbm.at[pl.ds(offt, 80)], rowt)
    pltpu.sync_copy(col_hbm.at[pl.ds(offt, 80)], colt)
    idx_math(rowt, colt, lrowt, eq80, 5)
    pltpu.sync_copy(ones80, acc_c.at[lrowt], add=True)
    pltpu.sync_copy(eq80, acc_l.at[lrowt], add=True)

    plsc.subcore_barrier()
    out_off = cid * RPAD + sid * TPC
    pltpu.sync_copy(acc_c.at[pl.ds(sid * TPC, TPC)], obuf)
    pltpu.sync_copy(obuf, cnt_hbm.at[pl.ds(out_off, TPC)])
    pltpu.sync_copy(acc_l.at[pl.ds(sid * TPC, TPC)], obuf)
    pltpu.sync_copy(obuf, lc_hbm.at[pl.ds(out_off, TPC)])


@functools.partial(
    pl.kernel,
    out_type=jax.ShapeDtypeStruct((NPAD, H), F32),
    mesh=_MESH,
    compiler_params=pltpu.CompilerParams(use_tc_tiling_on_sc=False),
    scratch_types=[
        pltpu.VMEM((128,), jnp.int32),
        pltpu.VMEM((128,), jnp.int32),
        pltpu.VMEM((128,), jnp.int32),
        pltpu.VMEM((128, H), F32),
        pltpu.VMEM((80,), jnp.int32),
        pltpu.VMEM((80,), jnp.int32),
        pltpu.VMEM((80,), jnp.int32),
        pltpu.VMEM((80, H), F32),
        pltpu.VMEM((16, H), F32),
        pltpu.VMEM((224, H), F32),
        pltpu.VMEM_SHARED((RPAD, H), F32),
        pltpu.SemaphoreType.DMA,
    ],
)
def _sc_conv(y_hbm, row_hbm, col_hbm, out_hbm,
             rowv, colv, lrowv, rows_v, rowt, colt, lrowt, rows_t,
             zb, obuf, acc, sem):
    cid = lax.axis_index("c")
    sid = lax.axis_index("s")
    base_n = cid * HALF
    for r in range(16):
        for k in range(4):
            zb[r, pl.ds(k * 16, 16)] = jnp.zeros((16,), F32)

    def zloop(i, carry):
        pltpu.sync_copy(zb, acc.at[pl.ds(sid * TPC + i * 16, 16)])
        return carry

    lax.fori_loop(0, TPC // 16, zloop, 0)
    plsc.subcore_barrier()

    base_e = sid * EPT

    def idx_math(rsrc, ldst, n16):
        for j in range(n16):
            rv = rsrc[pl.ds(j * 16, 16)]
            lv = rv - base_n
            oob = jnp.logical_or(lv < 0, lv >= HALF)
            ldst[pl.ds(j * 16, 16)] = jnp.where(oob, TRASH, lv)

    def chunk(ci, carry):
        off = base_e + ci * 128
        pltpu.sync_copy(row_hbm.at[pl.ds(off, 128)], rowv)
        pltpu.sync_copy(col_hbm.at[pl.ds(off, 128)], colv)
        pltpu.async_copy(y_hbm.at[colv], rows_v, sem).wait()
        idx_math(rowv, lrowv, 8)
        pltpu.sync_copy(rows_v, acc.at[lrowv], add=True)
        return carry

    lax.fori_loop(0, 390, chunk, 0)

    offt = base_e + 390 * 128
    pltpu.sync_copy(row_hbm.at[pl.ds(offt, 80)], rowt)
    pltpu.sync_copy(col_hbm.at[pl.ds(offt, 80)], colt)
    pltpu.async_copy(y_hbm.at[colt], rows_t, sem).wait()
    idx_math(rowt, lrowt, 5)
    pltpu.sync_copy(rows_t, acc.at[lrowt], add=True)

    plsc.subcore_barrier()
    out_off = cid * RPAD + sid * TPC

    def oloop(i, carry):
        pltpu.sync_copy(acc.at[pl.ds(sid * TPC + i * 224, 224)], obuf)
        pltpu.sync_copy(obuf, out_hbm.at[pl.ds(out_off + i * 224, 224)])
        return carry

    lax.fori_loop(0, 7, oloop, 0)


@functools.partial(
    pl.kernel,
    out_type=jax.ShapeDtypeStruct((E, H), F32),
    mesh=_MESH,
    compiler_params=pltpu.CompilerParams(use_tc_tiling_on_sc=False),
    scratch_types=[
        pltpu.VMEM((128,), jnp.int32),
        pltpu.VMEM((128,), jnp.int32),
        pltpu.VMEM((128, H), F32),
        pltpu.VMEM((128, H), F32),
        pltpu.VMEM((40,), jnp.int32),
        pltpu.VMEM((40,), jnp.int32),
        pltpu.VMEM((40, H), F32),
        pltpu.VMEM((40, H), F32),
        pltpu.SemaphoreType.DMA,
    ],
)
def _sc_edge(u_hbm, v_hbm, row_hbm, col_hbm, w_hbm,
             idxr, idxc, buf1, buf2, idxrt, idxct, buf1t, buf2t, sem):
    cid = lax.axis_index("c")
    sid = lax.axis_index("s")
    wid = sid * 2 + cid
    base_e = wid * 24960

    def chunk(ci, carry):
        off = base_e + ci * 128
        pltpu.sync_copy(row_hbm.at[pl.ds(off, 128)], idxr)
        pltpu.sync_copy(col_hbm.at[pl.ds(off, 128)], idxc)
        pltpu.async_copy(u_hbm.at[idxr], buf1, sem).wait()
        pltpu.async_copy(v_hbm.at[idxc], buf2, sem).wait()

        def radd(r, c2):
            for k in range(4):
                buf1[r, pl.ds(k * 16, 16)] = (buf1[r, pl.ds(k * 16, 16)]
                                              + buf2[r, pl.ds(k * 16, 16)])
            return c2

        lax.fori_loop(0, 128, radd, 0)
        pltpu.sync_copy(buf1, w_hbm.at[pl.ds(off, 128)])
        return carry

    lax.fori_loop(0, 195, chunk, 0)

    offt = 32 * 24960 + wid * 40
    pltpu.sync_copy(row_hbm.at[pl.ds(offt, 40)], idxrt)
    pltpu.sync_copy(col_hbm.at[pl.ds(offt, 40)], idxct)
    pltpu.async_copy(u_hbm.at[idxrt], buf1t, sem).wait()
    pltpu.async_copy(v_hbm.at[idxct], buf2t, sem).wait()

    def raddt(r, c2):
        for k in range(4):
            buf1t[r, pl.ds(k * 16, 16)] = (buf1t[r, pl.ds(k * 16, 16)]
                                           + buf2t[r, pl.ds(k * 16, 16)])
        return c2

    lax.fori_loop(0, 40, raddt, 0)
    pltpu.sync_copy(buf1t, w_hbm.at[pl.ds(offt, 40)])


def _prep_body(cnt_ref, lc_ref, dis_ref, lw_ref):
    cnt = cnt_ref[...]
    lc = lc_ref[...]
    lw = jnp.where(lc > 0.0, 0.0, 1.0)
    deg = cnt + lw
    dis = jnp.where(deg > 0.0, lax.rsqrt(jnp.maximum(deg, 1e-12)), 0.0)
    dis_ref[...] = dis
    lw_ref[...] = lw


def _tc_prep(cnt392, lc392):
    return pl.pallas_call(
        _prep_body,
        out_shape=(jax.ShapeDtypeStruct((392, 128), F32),
                   jax.ShapeDtypeStruct((392, 128), F32)),
    )(cnt392, lc392)


def _y1_body(x_ref, w_ref, dis_ref, y_ref):
    y_ref[...] = jnp.dot(x_ref[...], w_ref[...],
                         preferred_element_type=F32) * dis_ref[...]


def _tc_y1(xp, W1p, dis_n):
    blk = 1000
    return pl.pallas_call(
        _y1_body,
        grid=(N // blk,),
        in_specs=[pl.BlockSpec((blk, 128), lambda i: (i, 0)),
                  pl.BlockSpec((128, H), lambda i: (0, 0)),
                  pl.BlockSpec((blk, 1), lambda i: (i, 0))],
        out_specs=pl.BlockSpec((blk, H), lambda i: (i, 0)),
        out_shape=jax.ShapeDtypeStruct((N, H), F32),
    )(xp, W1p, dis_n)


def _fuse1_body(s_ref, y_ref, dis_ref, lw_ref, b_ref, w2_ref, h_ref, y2_ref):
    dis = dis_ref[...]
    h = jnp.maximum(dis * (s_ref[...] + lw_ref[...] * y_ref[...]) + b_ref[...], 0.0)
    h_ref[...] = h
    y2_ref[...] = jnp.dot(h, w2_ref[...], preferred_element_type=F32) * dis


def _tc_fuse1(s1, y1, dis_n, lw_n, b1r, W2):
    blk = 1000
    return pl.pallas_call(
        _fuse1_body,
        grid=(N // blk,),
        in_specs=[pl.BlockSpec((blk, H), lambda i: (i, 0)),
                  pl.BlockSpec((blk, H), lambda i: (i, 0)),
                  pl.BlockSpec((blk, 1), lambda i: (i, 0)),
                  pl.BlockSpec((blk, 1), lambda i: (i, 0)),
                  pl.BlockSpec((1, H), lambda i: (0, 0)),
                  pl.BlockSpec((H, H), lambda i: (0, 0))],
        out_specs=(pl.BlockSpec((blk, H), lambda i: (i, 0)),
                   pl.BlockSpec((blk, H), lambda i: (i, 0))),
        out_shape=(jax.ShapeDtypeStruct((N, H), F32),
                   jax.ShapeDtypeStruct((N, H), F32)),
    )(s1, y1, dis_n, lw_n, b1r, W2)


def _sigm(x):
    return 1.0 / (1.0 + jnp.exp(-x))


def _big_body(s2_ref, y2_ref, h1_ref, dis_ref, lw_ref, b2_ref,
              gw_ref, gb_ref, wf_ref, wb_ref,
              wlin_ref, blin_ref, wn1_ref, bn1_ref, wn2_ref, bn2_ref,
              we1a_ref, we1b_ref,
              nout_ref, u_ref, v_ref):
    dis = dis_ref[...]
    h1 = h1_ref[...]
    h2 = jnp.maximum(dis * (s2_ref[...] + lw_ref[...] * y2_ref[...]) + b2_ref[...], 0.0)

    gw = gw_ref[...]
    gb = gb_ref[...]

    def dot(a, b):
        return jnp.dot(a, b, preferred_element_type=F32)

    def cell(x, h, c, base, bbase, first):
        pre = [dot(x, gw[base + g]) + gb[bbase + g][None, :] for g in range(4)]
        if not first:
            for g in range(4):
                pre[g] = pre[g] + dot(h, gw[base + 4 + g])
        i = _sigm(pre[0]); f = _sigm(pre[1])
        g_ = jnp.tanh(pre[2]); o = _sigm(pre[3])
        c2 = (c * f if not first else 0.0) + i * g_
        h2_ = o * jnp.tanh(c2)
        return h2_, c2

    hf1, cf1 = cell(h1, None, None, 0, 0, True)
    hf2, _ = cell(h2, hf1, cf1, 0, 0, False)
    hb1, cb1 = cell(h2, None, None, 8, 4, True)
    hb2, _ = cell(h1, hb1, cb1, 8, 4, False)
    wf = wf_ref[...]
    wb = wb_ref[...]
    a0 = jnp.sum(hf1 * wf, axis=1, keepdims=True) + jnp.sum(hb2 * wb, axis=1, keepdims=True)
    a1 = jnp.sum(hf2 * wf, axis=1, keepdims=True) + jnp.sum(hb1 * wb, axis=1, keepdims=True)
    m = jnp.maximum(a0, a1)
    e0 = jnp.exp(a0 - m)
    e1 = jnp.exp(a1 - m)
    w0 = e0 / (e0 + e1)
    w1 = e1 / (e0 + e1)
    jk = w0 * h1 + w1 * h2
    hfin = dot(jk, wlin_ref[...]) + blin_ref[...]
    t = jnp.maximum(dot(hfin, wn1_ref[...]) + bn1_ref[...], 0.0)
    nlog = dot(t, wn2_ref[...]) + bn2_ref[...]
    nm = jnp.max(nlog, axis=1, keepdims=True)
    ne = jnp.exp(nlog - nm)
    nout_ref[...] = ne / jnp.sum(ne, axis=1, keepdims=True)
    u_ref[...] = dot(hfin, we1a_ref[...])
    v_ref[...] = dot(hfin, we1b_ref[...])


def _tc_big(s2, y2, h1, dis_n, lw_n, b2r, gw, gb, wf, wb,
            Wlin, blinr, Wn1, bn1r, Wn2p, bn2p, We1a, We1b):
    blk = 1000
    full = lambda shape: pl.BlockSpec(shape, lambda i: tuple(0 for _ in shape))
    row = lambda w: pl.BlockSpec((blk, w), lambda i: (i, 0))
    return pl.pallas_call(
        _big_body,
        grid=(N // blk,),
        in_specs=[row(H), row(H), row(H), row(1), row(1), full((1, H)),
                  full((16, H, H)), full((8, H)), full((1, H)), full((1, H)),
                  full((H, H)), full((1, H)), full((H, H)), full((1, H)),
                  full((H, 8)), full((1, 8)),
                  full((H, H)), full((H, H))],
        out_specs=(pl.BlockSpec((blk, 8), lambda i: (i, 0)), row(H), row(H)),
        out_shape=(jax.ShapeDtypeStruct((N, 8), F32),
                   jax.ShapeDtypeStruct((N, H), F32),
                   jax.ShapeDtypeStruct((N, H), F32)),
    )(s2, y2, h1, dis_n, lw_n, b2r, gw, gb, wf, wb,
      Wlin, blinr, Wn1, bn1r, Wn2p, bn2p, We1a, We1b)


def _edge_body(w_ref, be1_ref, we2_ref, be2_ref, out_ref):
    t = jnp.maximum(w_ref[...] + be1_ref[...], 0.0)
    lo = jnp.dot(t, we2_ref[...], preferred_element_type=F32) + be2_ref[...]
    m = jnp.max(lo, axis=1, keepdims=True)
    e = jnp.exp(lo - m)
    out_ref[...] = e / jnp.sum(e, axis=1, keepdims=True)


def _tc_edge(w, be1r, We2p, be2p):
    blk = 2000
    return pl.pallas_call(
        _edge_body,
        grid=(E // blk,),
        in_specs=[pl.BlockSpec((blk, H), lambda i: (i, 0)),
                  pl.BlockSpec((1, H), lambda i: (0, 0)),
                  pl.BlockSpec((H, 8), lambda i: (0, 0)),
                  pl.BlockSpec((1, 8), lambda i: (0, 0))],
        out_specs=pl.BlockSpec((blk, 8), lambda i: (i, 0)),
        out_shape=jax.ShapeDtypeStruct((E, 8), F32),
    )(w, be1r, We2p, be2p)


def _unpad1(a):
    return jnp.concatenate([a[:HALF], a[RPAD:RPAD + HALF]], axis=0)


def kernel(x, edge_index, W1, b1, W2, b2, Wih_f, Whh_f, bih_f, bhh_f,
           Wih_b, Whh_b, bih_b, bhh_b, Watt, batt, Wlin, blin,
           Wn1, bn1, Wn2, bn2, We1, be1, We2, be2):
    row0 = edge_index[0]
    col0 = edge_index[1]

    cnt, lc = _sc_deg(row0, col0)
    dis392, lw392 = _tc_prep(cnt.reshape(392, 128), lc.reshape(392, 128))
    dis_n = _unpad1(dis392.reshape(-1))[:, None]
    lw_n = _unpad1(lw392.reshape(-1))[:, None]

    xp = jnp.pad(x, ((0, 0), (0, 128 - x.shape[1])))
    W1p = jnp.pad(W1, ((0, 128 - W1.shape[0]), (0, 0)))
    y1 = _tc_y1(xp, W1p, dis_n)
    s1 = _unpad1(_sc_conv(y1, row0, col0))
    h1, y2 = _tc_fuse1(s1, y1, dis_n, lw_n, b1[None, :], W2)

    s2 = _unpad1(_sc_conv(y2, row0, col0))

    def gates_x(Wih):
        return [Wih[g * H:(g + 1) * H, :].T for g in range(4)]

    gw = jnp.stack(gates_x(Wih_f) + gates_x(Whh_f)
                   + gates_x(Wih_b) + gates_x(Whh_b))
    gbf = (bih_f + bhh_f).reshape(4, H)
    gbb = (bih_b + bhh_b).reshape(4, H)
    gb = jnp.concatenate([gbf, gbb], axis=0)
    wf = Watt[:H, 0][None, :]
    wb = Watt[H:, 0][None, :]
    NEG = jnp.float32(-1e30)
    Wn2p = jnp.pad(Wn2, ((0, 0), (0, 6)))
    bn2p = jnp.concatenate([bn2, jnp.full((6,), NEG)])[None, :]
    We1a = We1[:H, :]
    We1b = We1[H:, :]
    nout8, u, v = _tc_big(s2, y2, h1, dis_n, lw_n, b2[None, :],
                          gw, gb, wf, wb, Wlin, blin[None, :],
                          Wn1, bn1[None, :], Wn2p, bn2p, We1a, We1b)

    w = _sc_edge(u, v, row0, col0)
    We2p = jnp.pad(We2, ((0, 0), (0, 6)))
    be2p = jnp.concatenate([be2, jnp.full((6,), NEG)])[None, :]
    e8 = _tc_edge(w, be1[None, :], We2p, be2p)

    return nout8[:, :2], e8[:, :2]


# speedup vs baseline: 8.0625x; 1.2958x over previous
import functools

import jax
import jax.numpy as jnp
from jax import lax
from jax.experimental import pallas as pl
from jax.experimental.pallas import tpu as pltpu, tpu_sc as plsc

N = 50000
E = 800000
H = 64
HALF = 25000
RPAD = 25088
TRASH = 25000
NPAD = 2 * RPAD
TPC = 1568
EPT = E // 16
EPW = E // 32

_MESH = plsc.VectorSubcoreMesh(core_axis_name="c", subcore_axis_name="s",
                               num_cores=2, num_subcores=16)

F32 = jnp.float32


def _zero16(ref, n16):
    for j in range(n16):
        ref[pl.ds(j * 16, 16)] = jnp.zeros((16,), F32)


def _ones16(ref, n16):
    for j in range(n16):
        ref[pl.ds(j * 16, 16)] = jnp.full((16,), 1.0, F32)


@functools.partial(
    pl.kernel,
    out_type=[jax.ShapeDtypeStruct((NPAD,), F32),
              jax.ShapeDtypeStruct((NPAD,), F32)],
    mesh=_MESH,
    compiler_params=pltpu.CompilerParams(use_tc_tiling_on_sc=False),
    scratch_types=[
        pltpu.VMEM((5, 128), jnp.int32),
        pltpu.VMEM((5, 128), jnp.int32),
        pltpu.VMEM((5, 128), jnp.int32),
        pltpu.VMEM((5, 128), F32),
        pltpu.VMEM((80,), jnp.int32),
        pltpu.VMEM((80,), jnp.int32),
        pltpu.VMEM((80,), jnp.int32),
        pltpu.VMEM((128,), F32),
        pltpu.VMEM((80,), F32),
        pltpu.VMEM((80,), F32),
        pltpu.VMEM((16,), F32),
        pltpu.VMEM((TPC,), F32),
        pltpu.VMEM_SHARED((RPAD,), F32),
        pltpu.VMEM_SHARED((RPAD,), F32),
        pltpu.SemaphoreType.DMA,
        pltpu.SemaphoreType.DMA,
    ],
)
def _sc_deg(row_hbm, col_hbm, cnt_hbm, lc_hbm,
            rowm, colm, lrowm, eqm, rowt, colt, lrowt,
            ones128, ones80, eq80, zb, obuf, acc_c, acc_l, sem_i, sem_s):
    cid = lax.axis_index("c")
    sid = lax.axis_index("s")
    base_n = cid * HALF
    _ones16(ones128, 8)
    _ones16(ones80, 5)
    _zero16(zb, 1)

    def zloop(i, carry):
        off = sid * TPC + i * 16
        pltpu.sync_copy(zb, acc_c.at[pl.ds(off, 16)])
        pltpu.sync_copy(zb, acc_l.at[pl.ds(off, 16)])
        return carry

    lax.fori_loop(0, TPC // 16, zloop, 0)
    plsc.subcore_barrier()

    base_e = sid * EPT

    def sbody(sci, carry):
        off = base_e + sci * 640
        di = []
        for k in range(5):
            di.append(pltpu.async_copy(
                row_hbm.at[pl.ds(off + k * 128, 128)], rowm.at[k], sem_i))
            di.append(pltpu.async_copy(
                col_hbm.at[pl.ds(off + k * 128, 128)], colm.at[k], sem_i))
        for d in di:
            d.wait()
        for k in range(5):
            for j in range(8):
                rv = rowm[k, pl.ds(j * 16, 16)]
                cv = colm[k, pl.ds(j * 16, 16)]
                lv = rv - base_n
                oob = jnp.logical_or(lv < 0, lv >= HALF)
                lrowm[k, pl.ds(j * 16, 16)] = jnp.where(oob, TRASH, lv)
                eqm[k, pl.ds(j * 16, 16)] = jnp.where(
                    rv == cv, jnp.full((16,), 1.0, F32), jnp.zeros((16,), F32))
        dsn = []
        for k in range(5):
            dsn.append(pltpu.async_copy(ones128, acc_c.at[lrowm.at[k]],
                                        sem_s, add=True))
            dsn.append(pltpu.async_copy(eqm.at[k], acc_l.at[lrowm.at[k]],
                                        sem_s, add=True))
        for d in dsn:
            d.wait()
        return carry

    lax.fori_loop(0, 78, sbody, 0)

    offt = base_e + 78 * 640
    pltpu.sync_copy(row_hbm.at[pl.ds(offt, 80)], rowt)
    pltpu.sync_copy(col_hbm.at[pl.ds(offt, 80)], colt)
    for j in range(5):
        rv = rowt[pl.ds(j * 16, 16)]
        cv = colt[pl.ds(j * 16, 16)]
        lv = rv - base_n
        oob = jnp.logical_or(lv < 0, lv >= HALF)
        lrowt[pl.ds(j * 16, 16)] = jnp.where(oob, TRASH, lv)
        eq80[pl.ds(j * 16, 16)] = jnp.where(
            rv == cv, jnp.full((16,), 1.0, F32), jnp.zeros((16,), F32))
    pltpu.sync_copy(ones80, acc_c.at[lrowt], add=True)
    pltpu.sync_copy(eq80, acc_l.at[lrowt], add=True)

    plsc.subcore_barrier()
    out_off = cid * RPAD + sid * TPC
    pltpu.sync_copy(acc_c.at[pl.ds(sid * TPC, TPC)], obuf)
    pltpu.sync_copy(obuf, cnt_hbm.at[pl.ds(out_off, TPC)])
    pltpu.sync_copy(acc_l.at[pl.ds(sid * TPC, TPC)], obuf)
    pltpu.sync_copy(obuf, lc_hbm.at[pl.ds(out_off, TPC)])


@functools.partial(
    pl.kernel,
    out_type=jax.ShapeDtypeStruct((NPAD, H), F32),
    mesh=_MESH,
    compiler_params=pltpu.CompilerParams(use_tc_tiling_on_sc=False),
    scratch_types=[
        pltpu.VMEM((5, 64), jnp.int32),
        pltpu.VMEM((5, 64), jnp.int32),
        pltpu.VMEM((5, 64), jnp.int32),
        pltpu.VMEM((5, 64, H), F32),
        pltpu.VMEM((16, H), F32),
        pltpu.VMEM_SHARED((RPAD, H), F32),
        pltpu.SemaphoreType.DMA,
        pltpu.SemaphoreType.DMA,
        pltpu.SemaphoreType.DMA,
    ],
)
def _sc_conv(y_hbm, row_hbm, col_hbm, out_hbm,
             rowm, colm, lrowm, rows5, zb, acc, sem_i, sem_g, sem_s):
    cid = lax.axis_index("c")
    sid = lax.axis_index("s")
    base_n = cid * HALF
    for r in range(16):
        for k in range(4):
            zb[r, pl.ds(k * 16, 16)] = jnp.zeros((16,), F32)

    def zloop(i, carry):
        pltpu.sync_copy(zb, acc.at[pl.ds(sid * TPC + i * 16, 16)])
        return carry

    lax.fori_loop(0, TPC // 16, zloop, 0)
    plsc.subcore_barrier()

    extra = jnp.where(sid < 4, 1, 0)
    nsc = 156 + extra
    base_e = (sid * 156 + jnp.minimum(sid, 4)) * 320

    def sbody(sci, carry):
        off = base_e + sci * 320
        di = []
        for k in range(5):
            di.append(pltpu.async_copy(
                row_hbm.at[pl.ds(off + k * 64, 64)], rowm.at[k], sem_i))
            di.append(pltpu.async_copy(
                col_hbm.at[pl.ds(off + k * 64, 64)], colm.at[k], sem_i))
        for d in di:
            d.wait()
        dg = [pltpu.async_copy(y_hbm.at[colm.at[k]], rows5.at[k], sem_g)
              for k in range(5)]
        for k in range(5):
            for j in range(4):
                rv = rowm[k, pl.ds(j * 16, 16)]
                lv = rv - base_n
                oob = jnp.logical_or(lv < 0, lv >= HALF)
                lrowm[k, pl.ds(j * 16, 16)] = jnp.where(oob, TRASH, lv)
        for d in dg:
            d.wait()
        dsn = [pltpu.async_copy(rows5.at[k], acc.at[lrowm.at[k]], sem_s,
                                add=True)
               for k in range(5)]
        for d in dsn:
            d.wait()
        return carry

    lax.fori_loop(0, nsc, sbody, 0)

    plsc.subcore_barrier()
    out_off = cid * RPAD + sid * TPC
    pltpu.sync_copy(acc.at[pl.ds(sid * TPC, TPC)],
                    out_hbm.at[pl.ds(out_off, TPC)])


@functools.partial(
    pl.kernel,
    out_type=jax.ShapeDtypeStruct((E, H), F32),
    mesh=_MESH,
    compiler_params=pltpu.CompilerParams(use_tc_tiling_on_sc=False),
    scratch_types=[
        pltpu.VMEM((5, 128), jnp.int32),
        pltpu.VMEM((5, 128), jnp.int32),
        pltpu.VMEM((5, 128, H), F32),
        pltpu.VMEM((5, 128, H), F32),
        pltpu.VMEM((40,), jnp.int32),
        pltpu.VMEM((40,), jnp.int32),
        pltpu.VMEM((40, H), F32),
        pltpu.VMEM((40, H), F32),
        pltpu.SemaphoreType.DMA,
        pltpu.SemaphoreType.DMA,
        pltpu.SemaphoreType.DMA,
    ],
)
def _sc_edge(u_hbm, v_hbm, row_hbm, col_hbm, w_hbm,
             idxrm, idxcm, bufu, bufv, idxrt, idxct, buf1t, buf2t,
             sem_i, sem_g, sem_w):
    cid = lax.axis_index("c")
    sid = lax.axis_index("s")
    wid = sid * 2 + cid
    base_e = wid * 24960

    def sbody(sci, carry):
        off = base_e + sci * 640
        di = []
        for k in range(5):
            di.append(pltpu.async_copy(
                row_hbm.at[pl.ds(off + k * 128, 128)], idxrm.at[k], sem_i))
            di.append(pltpu.async_copy(
                col_hbm.at[pl.ds(off + k * 128, 128)], idxcm.at[k], sem_i))
        for d in di:
            d.wait()
        dg = []
        for k in range(5):
            dg.append(pltpu.async_copy(u_hbm.at[idxrm.at[k]], bufu.at[k],
                                       sem_g))
            dg.append(pltpu.async_copy(v_hbm.at[idxcm.at[k]], bufv.at[k],
                                       sem_g))
        for d in dg:
            d.wait()

        def radd(r, c2):
            for k in range(5):
                for c in range(4):
                    bufu[k, r, pl.ds(c * 16, 16)] = (
                        bufu[k, r, pl.ds(c * 16, 16)]
                        + bufv[k, r, pl.ds(c * 16, 16)])
            return c2

        lax.fori_loop(0, 128, radd, 0)
        dw = [pltpu.async_copy(bufu.at[k], w_hbm.at[pl.ds(off + k * 128, 128)],
                               sem_w)
              for k in range(5)]
        for d in dw:
            d.wait()
        return carry

    lax.fori_loop(0, 39, sbody, 0)

    offt = 32 * 24960 + wid * 40
    pltpu.sync_copy(row_hbm.at[pl.ds(offt, 40)], idxrt)
    pltpu.sync_copy(col_hbm.at[pl.ds(offt, 40)], idxct)
    pltpu.async_copy(u_hbm.at[idxrt], buf1t, sem_g).wait()
    pltpu.async_copy(v_hbm.at[idxct], buf2t, sem_g).wait()

    def raddt(r, c2):
        for k in range(4):
            buf1t[r, pl.ds(k * 16, 16)] = (buf1t[r, pl.ds(k * 16, 16)]
                                           + buf2t[r, pl.ds(k * 16, 16)])
        return c2

    lax.fori_loop(0, 40, raddt, 0)
    pltpu.sync_copy(buf1t, w_hbm.at[pl.ds(offt, 40)])


def _prep_body(cnt_ref, lc_ref, dis_ref, lw_ref):
    cnt = cnt_ref[...]
    lc = lc_ref[...]
    lw = jnp.where(lc > 0.0, 0.0, 1.0)
    deg = cnt + lw
    dis = jnp.where(deg > 0.0, lax.rsqrt(jnp.maximum(deg, 1e-12)), 0.0)
    dis_ref[...] = dis
    lw_ref[...] = lw


def _tc_prep(cnt392, lc392):
    return pl.pallas_call(
        _prep_body,
        out_shape=(jax.ShapeDtypeStruct((392, 128), F32),
                   jax.ShapeDtypeStruct((392, 128), F32)),
    )(cnt392, lc392)


def _y1_body(x_ref, w_ref, dis_ref, y_ref):
    y_ref[...] = jnp.dot(x_ref[...], w_ref[...],
                         preferred_element_type=F32) * dis_ref[...]


def _tc_y1(xp, W1p, dis_n):
    blk = 1000
    return pl.pallas_call(
        _y1_body,
        grid=(N // blk,),
        in_specs=[pl.BlockSpec((blk, 128), lambda i: (i, 0)),
                  pl.BlockSpec((128, H), lambda i: (0, 0)),
                  pl.BlockSpec((blk, 1), lambda i: (i, 0))],
        out_specs=pl.BlockSpec((blk, H), lambda i: (i, 0)),
        out_shape=jax.ShapeDtypeStruct((N, H), F32),
    )(xp, W1p, dis_n)


def _fuse1_body(s_ref, y_ref, dis_ref, lw_ref, b_ref, w2_ref, h_ref, y2_ref):
    dis = dis_ref[...]
    h = jnp.maximum(dis * (s_ref[...] + lw_ref[...] * y_ref[...]) + b_ref[...], 0.0)
    h_ref[...] = h
    y2_ref[...] = jnp.dot(h, w2_ref[...], preferred_element_type=F32) * dis


def _tc_fuse1(s1, y1, dis_n, lw_n, b1r, W2):
    blk = 1000
    return pl.pallas_call(
        _fuse1_body,
        grid=(N // blk,),
        in_specs=[pl.BlockSpec((blk, H), lambda i: (i, 0)),
                  pl.BlockSpec((blk, H), lambda i: (i, 0)),
                  pl.BlockSpec((blk, 1), lambda i: (i, 0)),
                  pl.BlockSpec((blk, 1), lambda i: (i, 0)),
                  pl.BlockSpec((1, H), lambda i: (0, 0)),
                  pl.BlockSpec((H, H), lambda i: (0, 0))],
        out_specs=(pl.BlockSpec((blk, H), lambda i: (i, 0)),
                   pl.BlockSpec((blk, H), lambda i: (i, 0))),
        out_shape=(jax.ShapeDtypeStruct((N, H), F32),
                   jax.ShapeDtypeStruct((N, H), F32)),
    )(s1, y1, dis_n, lw_n, b1r, W2)


def _sigm(x):
    return 1.0 / (1.0 + jnp.exp(-x))


def _big_body(s2_ref, y2_ref, h1_ref, dis_ref, lw_ref, b2_ref,
              gw_ref, gb_ref, wf_ref, wb_ref,
              wlin_ref, blin_ref, wn1_ref, bn1_ref, wn2_ref, bn2_ref,
              we1a_ref, we1b_ref,
              nout_ref, u_ref, v_ref):
    dis = dis_ref[...]
    h1 = h1_ref[...]
    h2 = jnp.maximum(dis * (s2_ref[...] + lw_ref[...] * y2_ref[...]) + b2_ref[...], 0.0)

    gw = gw_ref[...]
    gb = gb_ref[...]

    def dot(a, b):
        return jnp.dot(a, b, preferred_element_type=F32)

    def cell(x, h, c, base, bbase, first):
        pre = [dot(x, gw[base + g]) + gb[bbase + g][None, :] for g in range(4)]
        if not first:
            for g in range(4):
                pre[g] = pre[g] + dot(h, gw[base + 4 + g])
        i = _sigm(pre[0]); f = _sigm(pre[1])
        g_ = jnp.tanh(pre[2]); o = _sigm(pre[3])
        c2 = (c * f if not first else 0.0) + i * g_
        h2_ = o * jnp.tanh(c2)
        return h2_, c2

    hf1, cf1 = cell(h1, None, None, 0, 0, True)
    hf2, _ = cell(h2, hf1, cf1, 0, 0, False)
    hb1, cb1 = cell(h2, None, None, 8, 4, True)
    hb2, _ = cell(h1, hb1, cb1, 8, 4, False)
    wf = wf_ref[...]
    wb = wb_ref[...]
    a0 = jnp.sum(hf1 * wf, axis=1, keepdims=True) + jnp.sum(hb2 * wb, axis=1, keepdims=True)
    a1 = jnp.sum(hf2 * wf, axis=1, keepdims=True) + jnp.sum(hb1 * wb, axis=1, keepdims=True)
    m = jnp.maximum(a0, a1)
    e0 = jnp.exp(a0 - m)
    e1 = jnp.exp(a1 - m)
    w0 = e0 / (e0 + e1)
    w1 = e1 / (e0 + e1)
    jk = w0 * h1 + w1 * h2
    hfin = dot(jk, wlin_ref[...]) + blin_ref[...]
    t = jnp.maximum(dot(hfin, wn1_ref[...]) + bn1_ref[...], 0.0)
    nlog = dot(t, wn2_ref[...]) + bn2_ref[...]
    nm = jnp.max(nlog, axis=1, keepdims=True)
    ne = jnp.exp(nlog - nm)
    nout_ref[...] = ne / jnp.sum(ne, axis=1, keepdims=True)
    u_ref[...] = dot(hfin, we1a_ref[...])
    v_ref[...] = dot(hfin, we1b_ref[...])


def _tc_big(s2, y2, h1, dis_n, lw_n, b2r, gw, gb, wf, wb,
            Wlin, blinr, Wn1, bn1r, Wn2p, bn2p, We1a, We1b):
    blk = 1000
    full = lambda shape: pl.BlockSpec(shape, lambda i: tuple(0 for _ in shape))
    row = lambda w: pl.BlockSpec((blk, w), lambda i: (i, 0))
    return pl.pallas_call(
        _big_body,
        grid=(N // blk,),
        in_specs=[row(H), row(H), row(H), row(1), row(1), full((1, H)),
                  full((16, H, H)), full((8, H)), full((1, H)), full((1, H)),
                  full((H, H)), full((1, H)), full((H, H)), full((1, H)),
                  full((H, 8)), full((1, 8)),
                  full((H, H)), full((H, H))],
        out_specs=(pl.BlockSpec((blk, 8), lambda i: (i, 0)), row(H), row(H)),
        out_shape=(jax.ShapeDtypeStruct((N, 8), F32),
                   jax.ShapeDtypeStruct((N, H), F32),
                   jax.ShapeDtypeStruct((N, H), F32)),
    )(s2, y2, h1, dis_n, lw_n, b2r, gw, gb, wf, wb,
      Wlin, blinr, Wn1, bn1r, Wn2p, bn2p, We1a, We1b)


def _edge_body(w_ref, be1_ref, we2_ref, be2_ref, out_ref):
    t = jnp.maximum(w_ref[...] + be1_ref[...], 0.0)
    lo = jnp.dot(t, we2_ref[...], preferred_element_type=F32) + be2_ref[...]
    m = jnp.max(lo, axis=1, keepdims=True)
    e = jnp.exp(lo - m)
    out_ref[...] = e / jnp.sum(e, axis=1, keepdims=True)


def _tc_edge(w, be1r, We2p, be2p):
    blk = 2000
    return pl.pallas_call(
        _edge_body,
        grid=(E // blk,),
        in_specs=[pl.BlockSpec((blk, H), lambda i: (i, 0)),
                  pl.BlockSpec((1, H), lambda i: (0, 0)),
                  pl.BlockSpec((H, 8), lambda i: (0, 0)),
                  pl.BlockSpec((1, 8), lambda i: (0, 0))],
        out_specs=pl.BlockSpec((blk, 8), lambda i: (i, 0)),
        out_shape=jax.ShapeDtypeStruct((E, 8), F32),
    )(w, be1r, We2p, be2p)


def _unpad1(a):
    return jnp.concatenate([a[:HALF], a[RPAD:RPAD + HALF]], axis=0)


def kernel(x, edge_index, W1, b1, W2, b2, Wih_f, Whh_f, bih_f, bhh_f,
           Wih_b, Whh_b, bih_b, bhh_b, Watt, batt, Wlin, blin,
           Wn1, bn1, Wn2, bn2, We1, be1, We2, be2):
    row0 = edge_index[0]
    col0 = edge_index[1]

    cnt, lc = _sc_deg(row0, col0)
    dis392, lw392 = _tc_prep(cnt.reshape(392, 128), lc.reshape(392, 128))
    dis_n = _unpad1(dis392.reshape(-1))[:, None]
    lw_n = _unpad1(lw392.reshape(-1))[:, None]

    xp = jnp.pad(x, ((0, 0), (0, 128 - x.shape[1])))
    W1p = jnp.pad(W1, ((0, 128 - W1.shape[0]), (0, 0)))
    y1 = _tc_y1(xp, W1p, dis_n)
    s1 = _unpad1(_sc_conv(y1, row0, col0))
    h1, y2 = _tc_fuse1(s1, y1, dis_n, lw_n, b1[None, :], W2)

    s2 = _unpad1(_sc_conv(y2, row0, col0))

    def gates_x(Wih):
        return [Wih[g * H:(g + 1) * H, :].T for g in range(4)]

    gw = jnp.stack(gates_x(Wih_f) + gates_x(Whh_f)
                   + gates_x(Wih_b) + gates_x(Whh_b))
    gbf = (bih_f + bhh_f).reshape(4, H)
    gbb = (bih_b + bhh_b).reshape(4, H)
    gb = jnp.concatenate([gbf, gbb], axis=0)
    wf = Watt[:H, 0][None, :]
    wb = Watt[H:, 0][None, :]
    NEG = jnp.float32(-1e30)
    Wn2p = jnp.pad(Wn2, ((0, 0), (0, 6)))
    bn2p = jnp.concatenate([bn2, jnp.full((6,), NEG)])[None, :]
    We1a = We1[:H, :]
    We1b = We1[H:, :]
    nout8, u, v = _tc_big(s2, y2, h1, dis_n, lw_n, b2[None, :],
                          gw, gb, wf, wb, Wlin, blin[None, :],
                          Wn1, bn1[None, :], Wn2p, bn2p, We1a, We1b)

    w = _sc_edge(u, v, row0, col0)
    We2p = jnp.pad(We2, ((0, 0), (0, 6)))
    be2p = jnp.concatenate([be2, jnp.full((6,), NEG)])[None, :]
    e8 = _tc_edge(w, be1[None, :], We2p, be2p)

    return nout8[:, :2], e8[:, :2]
